# baseline (speedup 1.0000x reference)
import functools

import jax
import jax.numpy as jnp
from jax import lax
from jax.experimental import pallas as pl
from jax.experimental.pallas import tpu as pltpu
from jax.experimental.pallas import tpu_sc as plsc

H = 24
W = 24
HALF = 64
EMBED = 128
P = H * W
R = 2 * H - 1
NW = 32
PW = P // NW


def _build_body(tx_ref, ty_ref, out_ref):
    j = pl.program_id(0)
    cols = lax.broadcasted_iota(jnp.int32, (W, R), 1)
    rows = lax.broadcasted_iota(jnp.int32, (W, R), 0)
    onehot = jnp.where(cols == 23 - j + rows, 1.0, 0.0).astype(jnp.float32)
    shifted = jnp.dot(onehot, ty_ref[...], preferred_element_type=jnp.float32)
    out_ref[0, :, :, :HALF] = jnp.broadcast_to(
        tx_ref[...][:, None, :], (R, W, HALF)
    )
    out_ref[0, :, :, HALF:] = jnp.broadcast_to(shifted[None, :, :], (R, W, HALF))


def _sc_body(s_hbm, out_hbm, sem):
    c_id = lax.axis_index("c")
    s_id = lax.axis_index("s")
    wid = s_id * 2 + c_id
    base = wid * PW

    handles = []
    for t in range(PW):
        p = base + t
        ii = lax.div(p, W)
        jj = lax.rem(p, W)
        handles.append(
            pltpu.async_copy(
                s_hbm.at[jj, pl.ds(23 - ii, H)],
                out_hbm.at[p],
                sem,
            )
        )
    for hnd in handles:
        hnd.wait()


@functools.cache
def _sc_call():
    mesh = plsc.VectorSubcoreMesh(
        core_axis_name="c", subcore_axis_name="s", num_cores=2, num_subcores=16
    )
    return pl.kernel(
        _sc_body,
        out_type=jax.ShapeDtypeStruct((P, H, W, EMBED), jnp.float32),
        mesh=mesh,
        scratch_types=[
            pltpu.SemaphoreType.DMA,
        ],
    )


@jax.jit
def kernel(table_x, table_y):
    s_tab = pl.pallas_call(
        _build_body,
        grid=(W,),
        in_specs=[
            pl.BlockSpec((R, HALF), lambda d: (0, 0)),
            pl.BlockSpec((R, HALF), lambda d: (0, 0)),
        ],
        out_specs=pl.BlockSpec((1, R, W, EMBED), lambda d: (d, 0, 0, 0)),
        out_shape=jax.ShapeDtypeStruct((W, R, W, EMBED), jnp.float32),
    )(table_x, table_y)
    out128 = _sc_call()(s_tab)
    return out128.reshape(P, P, EMBED)


# speedup vs baseline: 32.9574x; 32.9574x over previous
import functools

import jax
import jax.numpy as jnp
from jax import lax
from jax.experimental import pallas as pl
from jax.experimental.pallas import tpu as pltpu
from jax.experimental.pallas import tpu_sc as plsc

H = 24
W = 24
HALF = 64
EMBED = 128
P = H * W
R = 2 * H - 1
NW = 32
PW = P // NW


def _build_body(tx_ref, ty_ref, out_ref):
    j = pl.program_id(0)
    cols = lax.broadcasted_iota(jnp.int32, (W, R), 1)
    rows = lax.broadcasted_iota(jnp.int32, (W, R), 0)
    onehot = jnp.where(cols == 23 - j + rows, 1.0, 0.0).astype(jnp.float32)
    shifted = jnp.dot(onehot, ty_ref[...], preferred_element_type=jnp.float32)
    out_ref[0, :, :, :HALF] = jnp.broadcast_to(
        tx_ref[...][:, None, :], (R, W, HALF)
    )
    out_ref[0, :, :, HALF:] = jnp.broadcast_to(shifted[None, :, :], (R, W, HALF))


HB = H // 2


def _sc_body(s_hbm, out_hbm, buf0_v, buf1_v, rsem, wsem):
    c_id = lax.axis_index("c")
    s_id = lax.axis_index("s")
    wid = s_id * 2 + c_id
    base = wid * PW

    bufs = [buf0_v, buf1_v]
    writes = [None, None]
    n = 0
    for t in range(PW):
        p = base + t
        ii = lax.div(p, W)
        jj = lax.rem(p, W)
        for h in range(2):
            b = n % 2
            if writes[b] is not None:
                writes[b].wait()
            rd = pltpu.async_copy(
                s_hbm.at[jj, pl.ds(23 - ii + h * HB, HB)], bufs[b], rsem
            )
            rd.wait()
            writes[b] = pltpu.async_copy(
                bufs[b], out_hbm.at[p, pl.ds(h * HB, HB)], wsem
            )
            n += 1
    for wr in writes:
        if wr is not None:
            wr.wait()


@functools.cache
def _sc_call():
    mesh = plsc.VectorSubcoreMesh(
        core_axis_name="c", subcore_axis_name="s", num_cores=2, num_subcores=16
    )
    return pl.kernel(
        _sc_body,
        out_type=jax.ShapeDtypeStruct((P, H, W, EMBED), jnp.float32),
        mesh=mesh,
        scratch_types=[
            pltpu.VMEM((HB, W, EMBED), jnp.float32),
            pltpu.VMEM((HB, W, EMBED), jnp.float32),
            pltpu.SemaphoreType.DMA,
            pltpu.SemaphoreType.DMA,
        ],
    )


@jax.jit
def kernel(table_x, table_y):
    s_tab = pl.pallas_call(
        _build_body,
        grid=(W,),
        in_specs=[
            pl.BlockSpec((R, HALF), lambda d: (0, 0)),
            pl.BlockSpec((R, HALF), lambda d: (0, 0)),
        ],
        out_specs=pl.BlockSpec((1, R, W, EMBED), lambda d: (d, 0, 0, 0)),
        out_shape=jax.ShapeDtypeStruct((W, R, W, EMBED), jnp.float32),
    )(table_x, table_y)
    out128 = _sc_call()(s_tab)
    return out128.reshape(P, P, EMBED)


# speedup vs baseline: 33.7721x; 1.0247x over previous
import functools

import jax
import jax.numpy as jnp
from jax import lax
from jax.experimental import pallas as pl
from jax.experimental.pallas import tpu as pltpu
from jax.experimental.pallas import tpu_sc as plsc

H = 24
W = 24
HALF = 64
EMBED = 128
P = H * W
R = 2 * H - 1
NW = 32
PW = P // NW


def _build_body(tx_ref, ty_ref, out_ref):
    j = pl.program_id(0)
    cols = lax.broadcasted_iota(jnp.int32, (W, R), 1)
    rows = lax.broadcasted_iota(jnp.int32, (W, R), 0)
    onehot = jnp.where(cols == 23 - j + rows, 1.0, 0.0).astype(jnp.float32)
    shifted = jnp.dot(onehot, ty_ref[...], preferred_element_type=jnp.float32)
    out_ref[0, :, :, :HALF] = jnp.broadcast_to(
        tx_ref[...][:, None, :], (R, W, HALF)
    )
    out_ref[0, :, :, HALF:] = jnp.broadcast_to(shifted[None, :, :], (R, W, HALF))


HB = H // 2
NBUF = 3
LAG = 2


def _sc_body(s_hbm, out_hbm, b0, b1, b2, rs0, rs1, rs2, ws0, ws1, ws2):
    c_id = lax.axis_index("c")
    s_id = lax.axis_index("s")
    wid = s_id * 2 + c_id
    base = wid * PW

    bufs = [b0, b1, b2]
    rsems = [rs0, rs1, rs2]
    wsems = [ws0, ws1, ws2]
    tasks = []
    for t in range(PW):
        p = base + t
        ii = lax.div(p, W)
        jj = lax.rem(p, W)
        for h in range(2):
            tasks.append((p, ii, jj, h))

    reads = [None] * NBUF
    writes = [None] * NBUF

    def issue_read(n):
        p, ii, jj, h = tasks[n]
        b = n % NBUF
        reads[b] = pltpu.async_copy(
            s_hbm.at[jj, pl.ds(23 - ii + h * HB, HB)], bufs[b], rsems[b]
        )

    def issue_write(n):
        p, _, _, h = tasks[n]
        b = n % NBUF
        reads[b].wait()
        writes[b] = pltpu.async_copy(
            bufs[b], out_hbm.at[p, pl.ds(h * HB, HB)], wsems[b]
        )

    ntask = len(tasks)
    for n in range(ntask):
        b = n % NBUF
        if writes[b] is not None:
            writes[b].wait()
        issue_read(n)
        if n >= LAG:
            issue_write(n - LAG)
    for n in range(ntask - LAG, ntask):
        issue_write(n)
    for wr in writes:
        wr.wait()


@functools.cache
def _sc_call():
    mesh = plsc.VectorSubcoreMesh(
        core_axis_name="c", subcore_axis_name="s", num_cores=2, num_subcores=16
    )
    return pl.kernel(
        _sc_body,
        out_type=jax.ShapeDtypeStruct((P, H, W, EMBED), jnp.float32),
        mesh=mesh,
        scratch_types=[
            pltpu.VMEM((HB, W, EMBED), jnp.float32),
            pltpu.VMEM((HB, W, EMBED), jnp.float32),
            pltpu.VMEM((HB, W, EMBED), jnp.float32),
            pltpu.SemaphoreType.DMA,
            pltpu.SemaphoreType.DMA,
            pltpu.SemaphoreType.DMA,
            pltpu.SemaphoreType.DMA,
            pltpu.SemaphoreType.DMA,
            pltpu.SemaphoreType.DMA,
        ],
    )


@jax.jit
def kernel(table_x, table_y):
    s_tab = pl.pallas_call(
        _build_body,
        grid=(W,),
        in_specs=[
            pl.BlockSpec((R, HALF), lambda d: (0, 0)),
            pl.BlockSpec((R, HALF), lambda d: (0, 0)),
        ],
        out_specs=pl.BlockSpec((1, R, W, EMBED), lambda d: (d, 0, 0, 0)),
        out_shape=jax.ShapeDtypeStruct((W, R, W, EMBED), jnp.float32),
    )(table_x, table_y)
    out128 = _sc_call()(s_tab)
    return out128.reshape(P, P, EMBED)


# speedup vs baseline: 53.8141x; 1.5934x over previous
import functools

import jax
import jax.numpy as jnp
from jax import lax
from jax.experimental import pallas as pl
from jax.experimental.pallas import tpu as pltpu
from jax.experimental.pallas import tpu_sc as plsc

H = 24
W = 24
HALF = 64
EMBED = 128
P = H * W
R = 2 * H - 1
NW = 32
PW = P // NW


def _build_body(tx_ref, ty_ref, out_ref):
    j = pl.program_id(0)
    cols = lax.broadcasted_iota(jnp.int32, (W, R), 1)
    rows = lax.broadcasted_iota(jnp.int32, (W, R), 0)
    onehot = jnp.where(cols == 23 - j + rows, 1.0, 0.0).astype(jnp.float32)
    shifted = jnp.dot(onehot, ty_ref[...], preferred_element_type=jnp.float32)
    out_ref[0, :, :, :HALF] = jnp.broadcast_to(
        tx_ref[...][:, None, :], (R, W, HALF)
    )
    out_ref[0, :, :, HALF:] = jnp.broadcast_to(shifted[None, :, :], (R, W, HALF))


G = 6
NG = H // G
NUNITS = W * NG
UPW = NUNITS // NW
WROWS = 23 + G


def _sc_body(s_hbm, out_hbm, buf_v, wsem):
    c_id = lax.axis_index("c")
    s_id = lax.axis_index("s")
    wid = s_id * 2 + c_id

    writes = []
    for m in range(UPW):
        u = wid * UPW + m
        jj = lax.div(u, NG)
        g = lax.rem(u, NG)
        i0 = g * G
        for wr in writes:
            wr.wait()
        writes = []
        pltpu.sync_copy(s_hbm.at[jj, pl.ds(18 - i0, WROWS)], buf_v)
        for d in range(G):
            p = (i0 + d) * W + jj
            writes.append(
                pltpu.async_copy(
                    buf_v.at[pl.ds(G - 1 - d, H)], out_hbm.at[p], wsem
                )
            )
    for wr in writes:
        wr.wait()


@functools.cache
def _sc_call():
    mesh = plsc.VectorSubcoreMesh(
        core_axis_name="c", subcore_axis_name="s", num_cores=2, num_subcores=16
    )
    return pl.kernel(
        _sc_body,
        out_type=jax.ShapeDtypeStruct((P, H, W, EMBED), jnp.float32),
        mesh=mesh,
        scratch_types=[
            pltpu.VMEM((WROWS, W, EMBED), jnp.float32),
            pltpu.SemaphoreType.DMA,
        ],
    )


@jax.jit
def kernel(table_x, table_y):
    s_tab = pl.pallas_call(
        _build_body,
        grid=(W,),
        in_specs=[
            pl.BlockSpec((R, HALF), lambda d: (0, 0)),
            pl.BlockSpec((R, HALF), lambda d: (0, 0)),
        ],
        out_specs=pl.BlockSpec((1, R, W, EMBED), lambda d: (d, 0, 0, 0)),
        out_shape=jax.ShapeDtypeStruct((W, R, W, EMBED), jnp.float32),
    )(table_x, table_y)
    out128 = _sc_call()(s_tab)
    return out128.reshape(P, P, EMBED)
